# baseline (speedup 1.0000x reference)
import functools

import jax
import jax.numpy as jnp
from jax import lax
from jax.experimental import pallas as pl
from jax.experimental.pallas import tpu as pltpu
from jax.experimental.pallas import tpu_sc as plsc

D = 128
SP = 56
NBUF = 8
PRE = 5


@functools.partial(jax.jit, static_argnames=("n_seq", "s", "nc", "ns"))
def _gather_sc(ids_flat, table, n_seq, s, nc, ns):
    nw = nc * ns
    per_w = n_seq // nw
    mesh = plsc.VectorSubcoreMesh(core_axis_name="c", subcore_axis_name="s")

    @functools.partial(
        pl.kernel,
        mesh=mesh,
        out_type=jax.ShapeDtypeStruct((n_seq, s, D), jnp.float32),
        scratch_types=(
            [pltpu.VMEM((per_w * SP,), jnp.int32),
             pltpu.VMEM((NBUF, SP, D), jnp.float32)]
            + [pltpu.SemaphoreType.DMA] * (2 * NBUF)
        ),
        compiler_params=pltpu.CompilerParams(use_tc_tiling_on_sc=True),
    )
    def k(ids_hbm, table_hbm, out_hbm, idx_v, rows_v, *sems):
        gsems, ssems = sems[:NBUF], sems[NBUF:]
        wid = lax.axis_index("s") * nc + lax.axis_index("c")
        base = wid * per_w
        pltpu.sync_copy(ids_hbm.at[pl.ds(base * SP, per_w * SP)], idx_v)

        def start_gather(j, b):
            pltpu.async_copy(table_hbm.at[idx_v.at[pl.ds(j * SP, SP)]],
                             rows_v.at[b], gsems[b])

        def wait_gather(b):
            pltpu.make_async_copy(
                table_hbm.at[idx_v.at[pl.ds(0, SP)]], rows_v.at[b],
                gsems[b]).wait()

        def start_store(j, b):
            pltpu.async_copy(rows_v.at[b, pl.ds(0, s)],
                             out_hbm.at[base + j], ssems[b])

        def wait_store(b):
            pltpu.make_async_copy(
                rows_v.at[b, pl.ds(0, s)], out_hbm.at[base], ssems[b]).wait()

        for j in range(PRE):
            start_gather(j, j % NBUF)

        def outer(i, carry):
            g = i * NBUF
            for b in range(NBUF):
                j = g + b
                wait_gather(b)
                start_store(j, b)
                jn = j + PRE
                bn = (b + PRE) % NBUF

                @pl.when(jn < per_w)
                def _():
                    @pl.when(jn >= NBUF)
                    def _():
                        wait_store(bn)
                    start_gather(jn, bn)

            return carry

        lax.fori_loop(0, per_w // NBUF, outer, 0)
        for b in range(NBUF):
            wait_store(b)

    return k(ids_flat, table)


def kernel(input_ids, table):
    b0, s = input_ids.shape
    info = plsc.get_sparse_core_info()
    nc, ns = info.num_cores, info.num_subcores
    ids = input_ids.astype(jnp.int32)
    ids_flat = jnp.pad(ids, ((0, 0), (0, SP - s))).reshape(b0 * SP)
    return _gather_sc(ids_flat, table, b0, s, nc, ns)


# speedup vs baseline: 2.2678x; 2.2678x over previous
import functools

import jax
import jax.numpy as jnp
from jax import lax
from jax.experimental import pallas as pl
from jax.experimental.pallas import tpu as pltpu
from jax.experimental.pallas import tpu_sc as plsc

D = 128
NBUF = 8
PRE = 5
K = 4
BS = 8


@functools.partial(jax.jit, static_argnames=("s", "nc", "ns"))
def _gather_sc(ids, table, s, nc, ns):
    nw = nc * ns
    n_seq = ids.shape[0]
    per_w = n_seq // nw
    mesh = plsc.VectorSubcoreMesh(core_axis_name="c", subcore_axis_name="s")

    @functools.partial(
        pl.kernel,
        mesh=mesh,
        out_type=jax.ShapeDtypeStruct((n_seq, s, D), jnp.float32),
        scratch_types=(
            [pltpu.VMEM((per_w, s), jnp.int32),
             pltpu.VMEM((NBUF, s, D), jnp.float32)]
            + [pltpu.SemaphoreType.DMA] * (2 * NBUF)
        ),
    )
    def k(ids_hbm, table_hbm, out_hbm, idx_v, rows_v, *sems):
        gsems, ssems = sems[:NBUF], sems[NBUF:]
        wid = lax.axis_index("s") * nc + lax.axis_index("c")
        base = wid * per_w
        pltpu.sync_copy(ids_hbm.at[pl.ds(base, per_w)], idx_v)

        def start_gather(j, b):
            pltpu.async_copy(table_hbm.at[idx_v.at[j]], rows_v.at[b], gsems[b])

        def wait_gather(b):
            pltpu.make_async_copy(
                table_hbm.at[idx_v.at[0]], rows_v.at[b], gsems[b]).wait()

        def start_store(j, b):
            pltpu.async_copy(rows_v.at[b], out_hbm.at[base + j], ssems[b])

        def wait_store(b):
            pltpu.make_async_copy(
                rows_v.at[b], out_hbm.at[base], ssems[b]).wait()

        for j in range(PRE):
            start_gather(j, j % NBUF)

        def outer(i, carry):
            g = i * NBUF
            for b in range(NBUF):
                j = g + b
                wait_gather(b)
                start_store(j, b)
                jn = j + PRE
                bn = (b + PRE) % NBUF

                @pl.when(jn < per_w)
                def _():
                    @pl.when(jn >= NBUF)
                    def _():
                        wait_store(bn)
                    start_gather(jn, bn)

            return carry

        lax.fori_loop(0, per_w // NBUF, outer, 0)
        for b in range(NBUF):
            wait_store(b)

    return k(ids, table)


def _relayout_body(in_ref, out_ref):
    for t in range(BS):
        out_ref[t] = in_ref[pl.ds(t * out_ref.shape[1], out_ref.shape[1])]


def _relayout_body_acc(in_ref, acc_ref, out_ref):
    del acc_ref
    _relayout_body(in_ref, out_ref)


def _relayout_tc(flat, acc, b0, s, blk0):
    grid = (flat.shape[0] // (BS * s),)
    in_specs = [pl.BlockSpec((BS * s, D), lambda j: (j, 0))]
    args = [flat]
    kwargs = {}
    body = _relayout_body
    if acc is not None:
        in_specs.append(pl.BlockSpec(memory_space=pl.ANY))
        args.append(acc)
        kwargs["input_output_aliases"] = {1: 0}
        body = _relayout_body_acc
    return pl.pallas_call(
        body,
        grid=grid,
        in_specs=in_specs,
        out_specs=pl.BlockSpec((BS, s, D), lambda j: (blk0 + j, 0, 0)),
        out_shape=jax.ShapeDtypeStruct((b0, s, D), jnp.float32),
        **kwargs,
    )(*args)


def kernel(input_ids, table):
    b0, s = input_ids.shape
    info = plsc.get_sparse_core_info()
    nc, ns = info.num_cores, info.num_subcores
    ids = input_ids.astype(jnp.int32)
    step = b0 // K
    flats = [
        _gather_sc(ids[k * step:(k + 1) * step], table, s, nc, ns)
        .reshape(step * s, D)
        for k in range(K)
    ]
    acc = None
    for k in range(K):
        acc = _relayout_tc(flats[k], acc, b0, s, k * (step // BS))
    return acc


# speedup vs baseline: 13.7349x; 6.0565x over previous
import functools

import jax
import jax.numpy as jnp
from jax import lax
from jax.experimental import pallas as pl
from jax.experimental.pallas import tpu as pltpu
from jax.experimental.pallas import tpu_sc as plsc

D = 128
C = 128
NBUF = 5
PRE = 3


@functools.partial(jax.jit, static_argnames=("nc", "ns"))
def _gather_sc(ids_blk, table, nc, ns):
    nw, s, _ = ids_blk.shape
    mesh = plsc.VectorSubcoreMesh(core_axis_name="c", subcore_axis_name="s")

    @functools.partial(
        pl.kernel,
        mesh=mesh,
        out_type=jax.ShapeDtypeStruct((s, nw * C, D), jnp.float32),
        scratch_types=(
            [pltpu.VMEM((s, C), jnp.int32),
             pltpu.VMEM((NBUF, C, D), jnp.float32)]
            + [pltpu.SemaphoreType.DMA] * (2 * NBUF)
        ),
    )
    def k(ids_hbm, table_hbm, out_hbm, idx_v, rows_v, *sems):
        gsems, ssems = sems[:NBUF], sems[NBUF:]
        wid = lax.axis_index("s") * nc + lax.axis_index("c")
        base = wid * C
        pltpu.sync_copy(ids_hbm.at[wid], idx_v)

        def start_gather(j, b):
            pltpu.async_copy(table_hbm.at[idx_v.at[j]], rows_v.at[b], gsems[b])

        def wait_gather(b):
            pltpu.make_async_copy(
                table_hbm.at[idx_v.at[0]], rows_v.at[b], gsems[b]).wait()

        def start_store(j, b):
            pltpu.async_copy(rows_v.at[b], out_hbm.at[j, pl.ds(base, C)],
                             ssems[b])

        def wait_store(b):
            pltpu.make_async_copy(
                rows_v.at[b], out_hbm.at[0, pl.ds(base, C)], ssems[b]).wait()

        for j in range(PRE):
            start_gather(j, j % NBUF)

        def outer(i, carry):
            g = i * NBUF
            for b in range(NBUF):
                j = g + b
                wait_gather(b)
                start_store(j, b)
                jn = j + PRE
                bn = (b + PRE) % NBUF

                @pl.when(jn < s)
                def _():
                    @pl.when(jn >= NBUF)
                    def _():
                        wait_store(bn)
                    start_gather(jn, bn)

            return carry

        lax.fori_loop(0, s // NBUF, outer, 0)
        for b in range(NBUF):
            wait_store(b)

    return k(ids_blk, table)


def kernel(input_ids, table):
    b0, s = input_ids.shape
    info = plsc.get_sparse_core_info()
    nc, ns = info.num_cores, info.num_subcores
    nw = nc * ns
    ids_blk = (input_ids.astype(jnp.int32)
               .T.reshape(s, nw, b0 // nw).transpose(1, 0, 2))
    out_t = _gather_sc(ids_blk, table, nc, ns)
    return jnp.transpose(out_t, (1, 0, 2))
